# baseline (speedup 1.0000x reference)
import functools

import jax
import jax.numpy as jnp
from jax import lax
from jax.experimental import pallas as pl
from jax.experimental.pallas import tpu as pltpu
from jax.experimental.pallas import tpu_sc as plsc

_D = 64
_SCALE = float(_D) ** 0.5
_NC = 2
_NS = 16
_NW = _NC * _NS
_LANES = 16
_NB = 4


@functools.partial(jax.jit, static_argnums=(2, 3))
def _lookup(x_flat, table, per_w, chunk):
  n_chunks = per_w // chunk
  assert n_chunks % _NB == 0
  mesh = plsc.VectorSubcoreMesh(
      core_axis_name="c", subcore_axis_name="s", num_cores=_NC,
      num_subcores=_NS)

  @functools.partial(
      pl.kernel,
      mesh=mesh,
      out_type=jax.ShapeDtypeStruct((x_flat.shape[0], _D), jnp.float32),
      scratch_types=[
          pltpu.VMEM((per_w,), jnp.int32),
          [pltpu.VMEM((chunk, _D), jnp.float32) for _ in range(_NB)],
          [pltpu.SemaphoreType.DMA for _ in range(_NB)],
          [pltpu.SemaphoreType.DMA for _ in range(_NB)],
      ],
      compiler_params=pltpu.CompilerParams(use_tc_tiling_on_sc=False),
  )
  def body(x_hbm, table_hbm, out_hbm, idx_v, rows, sem_g, sem_s):
    wid = lax.axis_index("s") * _NC + lax.axis_index("c")
    base = wid * per_w
    pltpu.sync_copy(x_hbm.at[pl.ds(base, per_w)], idx_v)

    def gather(c, b):
      return pltpu.make_async_copy(
          table_hbm.at[idx_v.at[pl.ds(c * chunk, chunk)]], rows[b], sem_g[b])

    def scatter(c, b):
      return pltpu.make_async_copy(
          rows[b], out_hbm.at[pl.ds(base + c * chunk, chunk)], sem_s[b])

    for b in range(_NB - 1):
      gather(b, b).start()

    def group_body(go, carry):
      for b in range(_NB):
        c = go * _NB + b
        gather(c, b).wait()

        def scale_body(r, acc):
          for j in range(_D // _LANES):
            sl = pl.ds(j * _LANES, _LANES)
            rows[b][r, sl] = rows[b][r, sl] * _SCALE
          return acc

        lax.fori_loop(0, chunk, scale_body, 0, unroll=4)
        scatter(c, b).start()

        nb = (b + _NB - 1) % _NB
        nc = c + _NB - 1

        @pl.when(nc < n_chunks)
        def _():
          @pl.when(c >= 1)
          def _():
            scatter(c - 1, nb).wait()
          gather(nc, nb).start()

      return carry

    lax.fori_loop(0, n_chunks // _NB, group_body, 0)
    for b in range(_NB):
      scatter(n_chunks - _NB + b, b).wait()

  return body(x_flat, table)


def kernel(x, table):
  b, l = x.shape
  n = b * l
  per_w = n // _NW
  x_flat = jnp.reshape(x, (n,)).astype(jnp.int32)
  out = _lookup(x_flat, table, per_w, 128)
  return jnp.reshape(out, (b, l, _D))
